# baseline (speedup 1.0000x reference)
import functools

import jax
import jax.numpy as jnp
from jax import lax
from jax.experimental import pallas as pl
from jax.experimental.pallas import tpu as pltpu
from jax.experimental.pallas import tpu_sc as plsc

_NUM_CORES = 2
_NUM_SUBCORES = 16
_NW = _NUM_CORES * _NUM_SUBCORES
_LANES = 16
_CH = 64
_NSLOT = 4


@functools.partial(jax.jit, static_argnames=("bb", "ll", "dp", "zrow"))
def _sc_gather(idx_flat, slen, table_p, bb, ll, dp, zrow):
    n_rows = bb * ll
    rpw = n_rows // _NW
    spw = bb // _NW
    n_chunks = rpw // _CH

    mesh = plsc.VectorSubcoreMesh(
        core_axis_name="c",
        subcore_axis_name="s",
        num_cores=_NUM_CORES,
        num_subcores=_NUM_SUBCORES,
    )

    @functools.partial(
        pl.kernel,
        out_type=jax.ShapeDtypeStruct((n_rows, dp), jnp.float32),
        mesh=mesh,
        compiler_params=pltpu.CompilerParams(use_tc_tiling_on_sc=False),
        scratch_types=[
            pltpu.VMEM((rpw,), jnp.int32),
            pltpu.VMEM((spw + _LANES,), jnp.int32),
            pltpu.VMEM((_NSLOT, _CH, dp), jnp.float32),
            [pltpu.SemaphoreType.DMA] * _NSLOT,
            [pltpu.SemaphoreType.DMA] * _NSLOT,
        ],
    )
    def run(idx_hbm, slen_hbm, table_hbm, out_hbm, idxv, slen_v, bufs, gsems, wsems):
        wid = lax.axis_index("s") * _NUM_CORES + lax.axis_index("c")
        base = wid * rpw
        pltpu.sync_copy(
            slen_hbm.at[pl.ds(wid * spw, spw)], slen_v.at[pl.ds(0, spw)]
        )
        pltpu.sync_copy(idx_hbm.at[pl.ds(base, rpw)], idxv)

        lane = lax.iota(jnp.int32, _LANES)
        zv = jnp.full((_LANES,), zrow, jnp.int32)
        offs = [j * _LANES for j in range(ll // _LANES)]
        if ll % _LANES:
            offs.append(ll - _LANES)

        def mask_seq(i, carry):
            n = slen_v[pl.ds(i, _LANES)][0]
            nv = jnp.full((_LANES,), n, jnp.int32)
            b = i * ll
            for off in offs:
                iv = idxv[pl.ds(b + off, _LANES)]
                idxv[pl.ds(b + off, _LANES)] = jnp.where(
                    lane + off < nv, iv, zv
                )
            return carry

        lax.fori_loop(0, spw, mask_seq, 0)

        def g_start(ci, slot):
            pltpu.async_copy(
                table_hbm.at[idxv.at[pl.ds(ci * _CH, _CH)]],
                bufs.at[slot],
                gsems[slot],
            )

        def g_wait(slot):
            pltpu.make_async_copy(
                table_hbm.at[idxv.at[pl.ds(0, _CH)]], bufs.at[slot], gsems[slot]
            ).wait()

        def w_start(ci, slot):
            pltpu.async_copy(
                bufs.at[slot], out_hbm.at[pl.ds(base + ci * _CH, _CH)], wsems[slot]
            )

        def w_drain(slot):
            pltpu.make_async_copy(
                bufs.at[slot], out_hbm.at[pl.ds(base, _CH)], wsems[slot]
            ).wait()

        g_start(0, 0)
        g_start(1, 1)
        g_start(2, 2)
        g_wait(0)
        w_start(0, 0)
        g_start(3, 3)
        g_wait(1)
        w_start(1, 1)
        w_drain(0)
        g_start(4, 0)
        g_wait(2)
        w_start(2, 2)
        w_drain(1)
        g_start(5, 1)
        g_wait(3)
        w_start(3, 3)

        def step(p, carry):
            for k in range(_NSLOT):
                ci = p * _NSLOT + k
                w_drain((k + 2) % _NSLOT)
                g_start(ci + 2, (k + 2) % _NSLOT)
                g_wait(k)
                w_start(ci, k)
            return carry

        lax.fori_loop(1, n_chunks // _NSLOT - 1, step, 0)

        pe = n_chunks - _NSLOT
        w_drain(2)
        g_start(pe + 2, 2)
        g_wait(0)
        w_start(pe, 0)
        w_drain(3)
        g_start(pe + 3, 3)
        g_wait(1)
        w_start(pe + 1, 1)
        g_wait(2)
        w_start(pe + 2, 2)
        g_wait(3)
        w_start(pe + 3, 3)
        w_drain(0)
        w_drain(1)
        w_drain(2)
        w_drain(3)

    return run(idx_flat, slen, table_p)


def kernel(indices, seq_lengths, table):
    bb, ll = indices.shape
    vv, dd = table.shape
    dp = (dd + _LANES - 1) // _LANES * _LANES
    idx_flat = indices.reshape(bb * ll).astype(jnp.int32)
    slen = seq_lengths.astype(jnp.int32)
    table_p = jnp.pad(table.astype(jnp.float32), ((0, 8), (0, dp - dd)))
    out = _sc_gather(idx_flat, slen, table_p, bb, ll, dp, vv)
    return out[:, :dd].reshape(bb, ll, dd)


# speedup vs baseline: 1.0015x; 1.0015x over previous
import functools

import jax
import jax.numpy as jnp
from jax import lax
from jax.experimental import pallas as pl
from jax.experimental.pallas import tpu as pltpu
from jax.experimental.pallas import tpu_sc as plsc

_NUM_CORES = 2
_NUM_SUBCORES = 16
_NW = _NUM_CORES * _NUM_SUBCORES
_LANES = 16
_CH = 128
_NSLOT = 3


def _mesh():
    return plsc.VectorSubcoreMesh(
        core_axis_name="c",
        subcore_axis_name="s",
        num_cores=_NUM_CORES,
        num_subcores=_NUM_SUBCORES,
    )


@functools.partial(jax.jit, static_argnames=("bb", "ll", "zrow"))
def _sc_mask(idx_flat, slen, bb, ll, zrow):
    n_rows = bb * ll
    rpw = n_rows // _NW
    spw = bb // _NW

    @functools.partial(
        pl.kernel,
        out_type=jax.ShapeDtypeStruct((n_rows,), jnp.int32),
        mesh=_mesh(),
        compiler_params=pltpu.CompilerParams(use_tc_tiling_on_sc=False),
        scratch_types=[
            pltpu.VMEM((rpw,), jnp.int32),
            pltpu.VMEM((spw + _LANES,), jnp.int32),
        ],
    )
    def run(idx_hbm, slen_hbm, out_hbm, idxv, slen_v):
        wid = lax.axis_index("s") * _NUM_CORES + lax.axis_index("c")
        base = wid * rpw
        pltpu.sync_copy(
            slen_hbm.at[pl.ds(wid * spw, spw)], slen_v.at[pl.ds(0, spw)]
        )
        pltpu.sync_copy(idx_hbm.at[pl.ds(base, rpw)], idxv)

        lane = lax.iota(jnp.int32, _LANES)
        zv = jnp.full((_LANES,), zrow, jnp.int32)
        offs = [j * _LANES for j in range(ll // _LANES)]
        if ll % _LANES:
            offs.append(ll - _LANES)

        def mask_seq(i, carry):
            n = slen_v[pl.ds(i, _LANES)][0]
            nv = jnp.full((_LANES,), n, jnp.int32)
            b = i * ll
            for off in offs:
                iv = idxv[pl.ds(b + off, _LANES)]
                idxv[pl.ds(b + off, _LANES)] = jnp.where(
                    lane + off < nv, iv, zv
                )
            return carry

        lax.fori_loop(0, spw, mask_seq, 0)
        pltpu.sync_copy(idxv, out_hbm.at[pl.ds(base, rpw)])

    return run(idx_flat, slen)


@functools.partial(jax.jit, static_argnames=("dp",))
def _sc_gather(midx2, table_p, dp):
    n_chunks, ch = midx2.shape
    n_rows = n_chunks * ch
    cpw = n_chunks // _NW

    @functools.partial(
        pl.kernel,
        out_type=jax.ShapeDtypeStruct((n_rows, dp), jnp.float32),
        mesh=_mesh(),
        compiler_params=pltpu.CompilerParams(use_tc_tiling_on_sc=False),
        scratch_types=[
            pltpu.VMEM((cpw, ch), jnp.int32),
            pltpu.VMEM((_NSLOT, ch, dp), jnp.float32),
            [pltpu.SemaphoreType.DMA] * _NSLOT,
            [pltpu.SemaphoreType.DMA] * _NSLOT,
        ],
    )
    def run(midx_hbm, table_hbm, out_hbm, idxv, bufs, gsems, wsems):
        wid = lax.axis_index("s") * _NUM_CORES + lax.axis_index("c")
        cbase = wid * cpw
        pltpu.sync_copy(midx_hbm.at[pl.ds(cbase, cpw)], idxv)

        def g_start(c, slot):
            pltpu.async_copy(
                table_hbm.at[idxv.at[c]], bufs.at[slot], gsems[slot]
            )

        def g_wait(slot):
            pltpu.make_async_copy(
                table_hbm.at[idxv.at[0]], bufs.at[slot], gsems[slot]
            ).wait()

        def w_start(c, slot):
            pltpu.async_copy(
                bufs.at[slot],
                out_hbm.at[pl.ds((cbase + c) * ch, ch)],
                wsems[slot],
            )

        def w_drain(slot):
            pltpu.make_async_copy(
                bufs.at[slot], out_hbm.at[pl.ds(0, ch)], wsems[slot]
            ).wait()

        g_start(0, 0)
        g_start(1, 1)
        g_wait(0)
        w_start(0, 0)
        g_start(2, 2)
        g_wait(1)
        w_start(1, 1)
        w_drain(0)
        g_start(3, 0)

        def step(p, carry):
            for j in range(3):
                c = 3 * p + 2 + j
                s = (2 + j) % _NSLOT
                g_wait(s)
                w_start(c, s)
                w_drain((j + 1) % _NSLOT)
                g_start(c + 2, (j + 1) % _NSLOT)
            return carry

        lax.fori_loop(0, (cpw - 5) // 3, step, 0)

        g_wait(2)
        w_start(cpw - 3, 2)
        w_drain(1)
        g_start(cpw - 1, 1)
        g_wait(0)
        w_start(cpw - 2, 0)
        w_drain(2)
        g_wait(1)
        w_start(cpw - 1, 1)
        w_drain(0)
        w_drain(1)

    return run(midx2, table_p)


def kernel(indices, seq_lengths, table):
    bb, ll = indices.shape
    vv, dd = table.shape
    dp = (dd + _LANES - 1) // _LANES * _LANES
    idx_flat = indices.reshape(bb * ll).astype(jnp.int32)
    slen = seq_lengths.astype(jnp.int32)
    table_p = jnp.pad(table.astype(jnp.float32), ((0, 8), (0, dp - dd)))
    midx = _sc_mask(idx_flat, slen, bb, ll, vv)
    midx2 = midx.reshape(bb * ll // _CH, _CH)
    out = _sc_gather(midx2, table_p, dp)
    return out[:, :dd].reshape(bb, ll, dd)


# speedup vs baseline: 1.0023x; 1.0008x over previous
import functools

import jax
import jax.numpy as jnp
from jax import lax
from jax.experimental import pallas as pl
from jax.experimental.pallas import tpu as pltpu
from jax.experimental.pallas import tpu_sc as plsc

_NUM_CORES = 2
_NUM_SUBCORES = 16
_NW = _NUM_CORES * _NUM_SUBCORES
_LANES = 16
_CH = 80
_NSLOT = 3


def _mesh():
    return plsc.VectorSubcoreMesh(
        core_axis_name="c",
        subcore_axis_name="s",
        num_cores=_NUM_CORES,
        num_subcores=_NUM_SUBCORES,
    )


@functools.partial(jax.jit, static_argnames=("bb", "ll", "zrow"))
def _sc_mask(idx_flat, slen, bb, ll, zrow):
    n_rows = bb * ll
    rpw = n_rows // _NW
    spw = bb // _NW

    @functools.partial(
        pl.kernel,
        out_type=jax.ShapeDtypeStruct((n_rows,), jnp.int32),
        mesh=_mesh(),
        compiler_params=pltpu.CompilerParams(use_tc_tiling_on_sc=False),
        scratch_types=[
            pltpu.VMEM((rpw,), jnp.int32),
            pltpu.VMEM((spw + _LANES,), jnp.int32),
        ],
    )
    def run(idx_hbm, slen_hbm, out_hbm, idxv, slen_v):
        wid = lax.axis_index("s") * _NUM_CORES + lax.axis_index("c")
        base = wid * rpw
        pltpu.sync_copy(
            slen_hbm.at[pl.ds(wid * spw, spw)], slen_v.at[pl.ds(0, spw)]
        )
        pltpu.sync_copy(idx_hbm.at[pl.ds(base, rpw)], idxv)

        lane = lax.iota(jnp.int32, _LANES)
        zv = jnp.full((_LANES,), zrow, jnp.int32)
        offs = [j * _LANES for j in range(ll // _LANES)]
        if ll % _LANES:
            offs.append(ll - _LANES)

        def mask_seq(i, carry):
            n = slen_v[pl.ds(i, _LANES)][0]
            nv = jnp.full((_LANES,), n, jnp.int32)
            b = i * ll
            for off in offs:
                iv = idxv[pl.ds(b + off, _LANES)]
                idxv[pl.ds(b + off, _LANES)] = jnp.where(
                    lane + off < nv, iv, zv
                )
            return carry

        lax.fori_loop(0, spw, mask_seq, 0)
        pltpu.sync_copy(idxv, out_hbm.at[pl.ds(base, rpw)])

    return run(idx_flat, slen)


@functools.partial(jax.jit, static_argnames=("dp",))
def _sc_gather(midx2, table_p, dp):
    n_chunks, ch = midx2.shape
    n_rows = n_chunks * ch
    cpw = n_chunks // _NW

    @functools.partial(
        pl.kernel,
        out_type=jax.ShapeDtypeStruct((n_rows, dp), jnp.float32),
        mesh=_mesh(),
        compiler_params=pltpu.CompilerParams(use_tc_tiling_on_sc=False),
        scratch_types=[
            pltpu.VMEM((cpw, ch), jnp.int32),
            pltpu.VMEM((_NSLOT, ch, dp), jnp.float32),
            [pltpu.SemaphoreType.DMA] * _NSLOT,
            [pltpu.SemaphoreType.DMA] * _NSLOT,
        ],
    )
    def run(midx_hbm, table_hbm, out_hbm, idxv, bufs, gsems, wsems):
        wid = lax.axis_index("s") * _NUM_CORES + lax.axis_index("c")
        cbase = wid * cpw
        pltpu.sync_copy(midx_hbm.at[pl.ds(cbase, cpw)], idxv)

        def g_start(c, slot):
            pltpu.async_copy(
                table_hbm.at[idxv.at[c]], bufs.at[slot], gsems[slot]
            )

        def g_wait(slot):
            pltpu.make_async_copy(
                table_hbm.at[idxv.at[0]], bufs.at[slot], gsems[slot]
            ).wait()

        def w_start(c, slot):
            pltpu.async_copy(
                bufs.at[slot],
                out_hbm.at[pl.ds((cbase + c) * ch, ch)],
                wsems[slot],
            )

        def w_drain(slot):
            pltpu.make_async_copy(
                bufs.at[slot], out_hbm.at[pl.ds(0, ch)], wsems[slot]
            ).wait()

        g_start(0, 0)
        g_start(1, 1)
        g_wait(0)
        w_start(0, 0)
        g_start(2, 2)
        g_wait(1)
        w_start(1, 1)
        w_drain(0)
        g_start(3, 0)

        def step(p, carry):
            for j in range(3):
                c = 3 * p + 2 + j
                s = (2 + j) % _NSLOT
                g_wait(s)
                w_start(c, s)
                w_drain((j + 1) % _NSLOT)
                g_start(c + 2, (j + 1) % _NSLOT)
            return carry

        lax.fori_loop(0, (cpw - 5) // 3, step, 0)

        g_wait(2)
        w_start(cpw - 3, 2)
        w_drain(1)
        g_start(cpw - 1, 1)
        g_wait(0)
        w_start(cpw - 2, 0)
        w_drain(2)
        g_wait(1)
        w_start(cpw - 1, 1)
        w_drain(0)
        w_drain(1)

    return run(midx2, table_p)


def kernel(indices, seq_lengths, table):
    bb, ll = indices.shape
    vv, dd = table.shape
    dp = (dd + 79) // 80 * 80
    idx_flat = indices.reshape(bb * ll).astype(jnp.int32)
    slen = seq_lengths.astype(jnp.int32)
    table_p = jnp.pad(table.astype(jnp.float32), ((0, 8), (0, dp - dd)))
    midx = _sc_mask(idx_flat, slen, bb, ll, vv)
    midx2 = midx.reshape(bb * ll // _CH, _CH)
    out = _sc_gather(midx2, table_p, dp)
    return out[:, :dd].reshape(bb, ll, dd)
